# Initial kernel scaffold; baseline (speedup 1.0000x reference)
import jax
import jax.numpy as jnp
from jax.experimental import pallas as pl


def kernel(x, input_ids, attention_mask, Wg, W1, b1, W2, b2):
    raise NotImplementedError("write your pallas kernel here")


# speedup vs baseline: 2.2151x; 2.2151x over previous
import functools

import jax
import jax.numpy as jnp
from jax import lax
from jax.experimental import pallas as pl
from jax.experimental.pallas import tpu as pltpu
from jax.experimental.pallas import tpu_sc as plsc

E = 8
D = 1024
DFF = 4096
N = 2048
TB = 1024
SUB = 128
NSUB = TB // SUB
NB = 9
NPAD = NB * TB
DK = 512
K = DFF // DK

NW = 32
CHUNK = N // NW


def _router_body(x_ref, wg_ref, pos_ref, psel_ref, counts_ref, be_ref,
                 sa_ref, bal_ref):
    x = x_ref[...]
    wg = wg_ref[...]
    logits = lax.dot_general(x, wg, (((1,), (1,)), ((), ())),
                             preferred_element_type=jnp.float32)
    m = jnp.max(logits, axis=1, keepdims=True)
    ex = jnp.exp(logits - m)
    probs = ex / jnp.sum(ex, axis=1, keepdims=True)
    pm = jnp.max(probs, axis=1, keepdims=True)
    eids = lax.broadcasted_iota(jnp.int32, (N, E), 1)
    gate = jnp.min(jnp.where(probs >= pm, eids, E), axis=1, keepdims=True)
    onehot = (eids == gate).astype(jnp.int32)
    counts = jnp.sum(onehot, axis=0, keepdims=True)

    c = onehot
    sh = 1
    while sh < N:
        shifted = jnp.concatenate(
            [jnp.zeros((sh, E), jnp.int32), c[:N - sh, :]], axis=0)
        c = c + shifted
        sh *= 2
    rank = jnp.sum(onehot * c, axis=1, keepdims=True) - 1

    blk_cnt = (counts + (TB - 1)) // TB
    er = lax.broadcasted_iota(jnp.int32, (E, E), 0)
    ec = lax.broadcasted_iota(jnp.int32, (E, E), 1)
    lt = (er < ec).astype(jnp.int32)
    blk_start = jnp.sum(
        jnp.broadcast_to(blk_cnt.reshape(E, 1), (E, E)) * lt,
        axis=0, keepdims=True)
    row_start = blk_start * TB
    pos = jnp.sum(onehot * row_start, axis=1, keepdims=True) + rank
    pos_ref[...] = pos
    psel_ref[...] = pm
    counts_ref[...] = counts

    total_blocks = jnp.sum(blk_cnt)
    bid = lax.broadcasted_iota(jnp.int32, (NB, E), 0)
    bs_b = jnp.broadcast_to(blk_start, (NB, E))
    be = jnp.sum((bs_b <= bid).astype(jnp.int32), axis=1,
                 keepdims=True) - 1
    bec = jnp.clip(be, 0, E - 1)
    be_ref[...] = bec

    esel = (lax.broadcasted_iota(jnp.int32, (NB, E), 1) == bec)
    bsel = jnp.sum(jnp.where(esel, bs_b, 0), axis=1, keepdims=True)
    csel = jnp.sum(jnp.where(esel, jnp.broadcast_to(counts, (NB, E)), 0),
                   axis=1, keepdims=True)
    bid1 = lax.broadcasted_iota(jnp.int32, (NB, 1), 0)
    rem = jnp.clip(csel - (bid1 - bsel) * TB, 0, TB)
    sa = (rem + (SUB - 1)) // SUB
    sa_ref[...] = jnp.where(bid1 < total_blocks, sa, 0)

    P = jnp.mean(probs, axis=0, keepdims=True)
    f = counts.astype(jnp.float32) / jnp.float32(N)
    bal_ref[...] = (jnp.float32(E) * jnp.sum(P * f)).reshape(1, 1)


def _router(x2d, wg):
    return pl.pallas_call(
        _router_body,
        out_shape=(
            jax.ShapeDtypeStruct((N, 1), jnp.int32),
            jax.ShapeDtypeStruct((N, 1), jnp.float32),
            jax.ShapeDtypeStruct((1, E), jnp.int32),
            jax.ShapeDtypeStruct((NB, 1), jnp.int32),
            jax.ShapeDtypeStruct((NB, 1), jnp.int32),
            jax.ShapeDtypeStruct((1, 1), jnp.float32),
        ),
    )(x2d, wg)


def _sc_wid():
    return lax.axis_index("s") * 2 + lax.axis_index("c")


def _dispatch_body(x_hbm, pos_hbm, psel_hbm, xs_hbm, ps_hbm,
                   idx_v, rows_v, pv_v, sem):
    base = _sc_wid() * CHUNK
    pltpu.sync_copy(pos_hbm.at[pl.ds(base, CHUNK)], idx_v)
    pltpu.sync_copy(x_hbm.at[pl.ds(base, CHUNK)], rows_v)
    pltpu.sync_copy(psel_hbm.at[pl.ds(base, CHUNK)], pv_v)
    pltpu.async_copy(rows_v, xs_hbm.at[idx_v], sem).wait()
    pltpu.async_copy(pv_v, ps_hbm.at[idx_v], sem).wait()


@functools.cache
def _get_dispatch():
    return pl.kernel(
        _dispatch_body,
        mesh=plsc.VectorSubcoreMesh(core_axis_name="c", subcore_axis_name="s",
                                    num_cores=2, num_subcores=16),
        out_type=(
            jax.ShapeDtypeStruct((NPAD, D), jnp.float32),
            jax.ShapeDtypeStruct((NPAD,), jnp.float32),
        ),
        scratch_types=[
            pltpu.VMEM((CHUNK,), jnp.int32),
            pltpu.VMEM((CHUNK, D), jnp.float32),
            pltpu.VMEM((CHUNK,), jnp.float32),
            pltpu.SemaphoreType.DMA,
        ],
    )


def _ffn_body(be_ref, sa_ref, x_ref, w1_ref, b1_ref, w2_ref, b2_ref, ps_ref,
              o_ref):
    b = pl.program_id(0)
    k = pl.program_id(1)
    sa = sa_ref[b]

    @pl.when(k == 0)
    def _():
        o_ref[...] = jnp.zeros_like(o_ref)

    for s in range(NSUB):
        @pl.when(s < sa)
        def _(s=s):
            rows = pl.ds(s * SUB, SUB)
            h = jnp.dot(x_ref[rows, :], w1_ref[0],
                        preferred_element_type=jnp.float32) + b1_ref[0]
            h = jax.nn.gelu(h)
            o_ref[rows, :] += jnp.dot(h, w2_ref[0],
                                      preferred_element_type=jnp.float32)

    @pl.when(k == K - 1)
    def _():
        o_ref[...] = (o_ref[...] + b2_ref[0]) * ps_ref[...]


def _ffn(be, sa, xs, w1, b1, w2, b2, ps2d):
    grid_spec = pltpu.PrefetchScalarGridSpec(
        num_scalar_prefetch=2,
        grid=(NB, K),
        in_specs=[
            pl.BlockSpec((TB, D), lambda b, k, be, sa: (b, 0)),
            pl.BlockSpec((1, D, DK), lambda b, k, be, sa: (be[b], 0, k)),
            pl.BlockSpec((1, 1, DK),
                         lambda b, k, be, sa: (be[b] * K + k, 0, 0)),
            pl.BlockSpec((1, DK, D), lambda b, k, be, sa: (be[b], k, 0)),
            pl.BlockSpec((1, 1, D), lambda b, k, be, sa: (be[b], 0, 0)),
            pl.BlockSpec((TB, 1), lambda b, k, be, sa: (b, 0)),
        ],
        out_specs=pl.BlockSpec((TB, D), lambda b, k, be, sa: (b, 0)),
    )
    return pl.pallas_call(
        _ffn_body,
        grid_spec=grid_spec,
        out_shape=jax.ShapeDtypeStruct((NPAD, D), jnp.float32),
        compiler_params=pltpu.CompilerParams(
            dimension_semantics=("arbitrary", "arbitrary")),
    )(be, sa, xs, w1, b1.reshape(E * K, 1, DK), w2, b2.reshape(E, 1, D),
      ps2d)


def _combine_body(ys_hbm, pos_hbm, out_hbm, idx_v, rows_v, sem):
    base = _sc_wid() * CHUNK
    pltpu.sync_copy(pos_hbm.at[pl.ds(base, CHUNK)], idx_v)
    pltpu.async_copy(ys_hbm.at[idx_v], rows_v, sem).wait()
    pltpu.sync_copy(rows_v, out_hbm.at[pl.ds(base, CHUNK)])


@functools.cache
def _get_combine():
    return pl.kernel(
        _combine_body,
        mesh=plsc.VectorSubcoreMesh(core_axis_name="c", subcore_axis_name="s",
                                    num_cores=2, num_subcores=16),
        out_type=jax.ShapeDtypeStruct((N, D), jnp.float32),
        scratch_types=[
            pltpu.VMEM((CHUNK,), jnp.int32),
            pltpu.VMEM((CHUNK, D), jnp.float32),
            pltpu.SemaphoreType.DMA,
        ],
    )


@jax.jit
def kernel(x, input_ids, attention_mask, Wg, W1, b1, W2, b2):
    del input_ids, attention_mask
    bsz, seq_len, dim = x.shape
    x2d = x.reshape(N, D)

    pos2d, psel2d, counts, be2d, sa2d, bal = _router(x2d, Wg)
    pos = pos2d.reshape(N)
    xs, ps = _get_dispatch()(x2d, pos, psel2d.reshape(N))
    ys = _ffn(be2d.reshape(NB), sa2d.reshape(NB), xs, W1, b1, W2, b2,
              ps.reshape(NPAD, 1))
    out2d = _get_combine()(ys, pos)

    out = out2d.reshape(bsz, seq_len, dim)
    balance_loss = bal.reshape(())
    gate_load = counts.reshape(E)
    return out, balance_loss, gate_load


# speedup vs baseline: 3.4701x; 1.5666x over previous
import functools

import jax
import jax.numpy as jnp
from jax import lax
from jax.experimental import pallas as pl
from jax.experimental.pallas import tpu as pltpu
from jax.experimental.pallas import tpu_sc as plsc

E = 8
D = 1024
DFF = 4096
N = 2048
TB = 512
SUB = 128
NSUB = TB // SUB
NB = 11
NPAD = NB * TB
DK = 2048
K = DFF // DK

NW = 32
CHUNK = N // NW


def _router_body(x_ref, wg_ref, pos_ref, psel_ref, counts_ref, bmap_ref,
                 be_ref, sa_ref, bal_ref):
    x = x_ref[...]
    wg = wg_ref[...]
    logits = lax.dot_general(x, wg, (((1,), (1,)), ((), ())),
                             preferred_element_type=jnp.float32)
    m = jnp.max(logits, axis=1, keepdims=True)
    ex = jnp.exp(logits - m)
    probs = ex / jnp.sum(ex, axis=1, keepdims=True)
    pm = jnp.max(probs, axis=1, keepdims=True)
    eids = lax.broadcasted_iota(jnp.int32, (N, E), 1)
    gate = jnp.min(jnp.where(probs >= pm, eids, E), axis=1, keepdims=True)
    onehot = (eids == gate).astype(jnp.int32)
    counts = jnp.sum(onehot, axis=0, keepdims=True)

    c = onehot
    sh = 1
    while sh < N:
        shifted = jnp.concatenate(
            [jnp.zeros((sh, E), jnp.int32), c[:N - sh, :]], axis=0)
        c = c + shifted
        sh *= 2
    rank = jnp.sum(onehot * c, axis=1, keepdims=True) - 1

    blk_cnt = (counts + (TB - 1)) // TB
    er = lax.broadcasted_iota(jnp.int32, (E, E), 0)
    ec = lax.broadcasted_iota(jnp.int32, (E, E), 1)
    lt = (er < ec).astype(jnp.int32)
    blk_start = jnp.sum(
        jnp.broadcast_to(blk_cnt.reshape(E, 1), (E, E)) * lt,
        axis=0, keepdims=True)
    row_start = blk_start * TB
    pos = jnp.sum(onehot * row_start, axis=1, keepdims=True) + rank
    pos_ref[...] = pos
    psel_ref[...] = jnp.broadcast_to(pm, (N, 16))
    counts_ref[...] = counts

    total_blocks = jnp.sum(blk_cnt)
    bid1 = lax.broadcasted_iota(jnp.int32, (NB, 1), 0)
    bmap = jnp.minimum(bid1, total_blocks - 1)
    bmap_ref[...] = bmap
    bidc = jnp.broadcast_to(bmap, (NB, E))
    bs_b = jnp.broadcast_to(blk_start, (NB, E))
    be = jnp.sum((bs_b <= bidc).astype(jnp.int32), axis=1,
                 keepdims=True) - 1
    bec = jnp.clip(be, 0, E - 1)
    be_ref[...] = bec

    esel = (lax.broadcasted_iota(jnp.int32, (NB, E), 1) == bec)
    bsel = jnp.sum(jnp.where(esel, bs_b, 0), axis=1, keepdims=True)
    csel = jnp.sum(jnp.where(esel, jnp.broadcast_to(counts, (NB, E)), 0),
                   axis=1, keepdims=True)
    rem = jnp.clip(csel - (bid1 - bsel) * TB, 0, TB)
    sa = (rem + (SUB - 1)) // SUB
    sa_ref[...] = jnp.where(bid1 < total_blocks, sa, 0)

    P = jnp.mean(probs, axis=0, keepdims=True)
    f = counts.astype(jnp.float32) / jnp.float32(N)
    bal_ref[...] = (jnp.float32(E) * jnp.sum(P * f)).reshape(1, 1)


def _router(x2d, wg):
    return pl.pallas_call(
        _router_body,
        out_shape=(
            jax.ShapeDtypeStruct((N, 1), jnp.int32),
            jax.ShapeDtypeStruct((N, 16), jnp.float32),
            jax.ShapeDtypeStruct((1, E), jnp.int32),
            jax.ShapeDtypeStruct((NB, 1), jnp.int32),
            jax.ShapeDtypeStruct((NB, 1), jnp.int32),
            jax.ShapeDtypeStruct((NB, 1), jnp.int32),
            jax.ShapeDtypeStruct((1, 1), jnp.float32),
        ),
    )(x2d, wg)


def _sc_wid():
    return lax.axis_index("s") * 2 + lax.axis_index("c")


HALF = CHUNK // 2


def _dispatch_body(x_hbm, pos_hbm, xs_hbm,
                   idx0, idx1, rows0, rows1, semA, semB):
    base = _sc_wid() * CHUNK
    pltpu.sync_copy(pos_hbm.at[pl.ds(base, HALF)], idx0)
    pltpu.sync_copy(pos_hbm.at[pl.ds(base + HALF, HALF)], idx1)
    r0 = pltpu.async_copy(x_hbm.at[pl.ds(base, HALF)], rows0, semA)
    r1 = pltpu.async_copy(x_hbm.at[pl.ds(base + HALF, HALF)], rows1, semB)
    r0.wait()
    s0 = pltpu.async_copy(rows0, xs_hbm.at[idx0], semA)
    r1.wait()
    s1 = pltpu.async_copy(rows1, xs_hbm.at[idx1], semB)
    s0.wait()
    s1.wait()


@functools.cache
def _get_dispatch():
    return pl.kernel(
        _dispatch_body,
        mesh=plsc.VectorSubcoreMesh(core_axis_name="c", subcore_axis_name="s",
                                    num_cores=2, num_subcores=16),
        out_type=jax.ShapeDtypeStruct((NPAD, D), jnp.float32),
        scratch_types=[
            pltpu.VMEM((HALF,), jnp.int32),
            pltpu.VMEM((HALF,), jnp.int32),
            pltpu.VMEM((HALF, D), jnp.float32),
            pltpu.VMEM((HALF, D), jnp.float32),
            pltpu.SemaphoreType.DMA,
            pltpu.SemaphoreType.DMA,
        ],
    )


def _ffn_body(bm_ref, be_ref, sa_ref, x_ref, w1_ref, b1_ref, w2_ref, b2_ref,
              o_ref):
    b = pl.program_id(0)
    k = pl.program_id(1)
    sa = sa_ref[b]

    @pl.when((k == 0) & (sa > 0))
    def _():
        o_ref[...] = jnp.zeros_like(o_ref)

    for s in range(NSUB):
        @pl.when(s < sa)
        def _(s=s):
            rows = pl.ds(s * SUB, SUB)
            h = jnp.dot(x_ref[rows, :], w1_ref[0],
                        preferred_element_type=jnp.float32) + b1_ref[0]
            h = jax.nn.gelu(h)
            o_ref[rows, :] += jnp.dot(h, w2_ref[0],
                                      preferred_element_type=jnp.float32)

    @pl.when((k == K - 1) & (sa > 0))
    def _():
        o_ref[...] = o_ref[...] + b2_ref[0]


def _ffn(bm, be, sa, xs, w1, b1, w2, b2):
    def kk(b, k, bm):
        return jnp.where(bm[b] == b, k, K - 1)

    grid_spec = pltpu.PrefetchScalarGridSpec(
        num_scalar_prefetch=3,
        grid=(NB, K),
        in_specs=[
            pl.BlockSpec((TB, D), lambda b, k, bm, be, sa: (bm[b], 0)),
            pl.BlockSpec((1, D, DK),
                         lambda b, k, bm, be, sa: (be[b], 0, kk(b, k, bm))),
            pl.BlockSpec((1, 1, DK),
                         lambda b, k, bm, be, sa:
                         (be[b] * K + kk(b, k, bm), 0, 0)),
            pl.BlockSpec((1, DK, D),
                         lambda b, k, bm, be, sa: (be[b], kk(b, k, bm), 0)),
            pl.BlockSpec((1, 1, D), lambda b, k, bm, be, sa: (be[b], 0, 0)),
        ],
        out_specs=pl.BlockSpec((TB, D), lambda b, k, bm, be, sa: (bm[b], 0)),
    )
    return pl.pallas_call(
        _ffn_body,
        grid_spec=grid_spec,
        out_shape=jax.ShapeDtypeStruct((NPAD, D), jnp.float32),
        compiler_params=pltpu.CompilerParams(
            dimension_semantics=("arbitrary", "arbitrary")),
    )(bm, be, sa, xs, w1, b1.reshape(E * K, 1, DK), w2, b2.reshape(E, 1, D))


def _scale_rows(rows, pv_v, off):
    def body(r, carry):
        pvec = pv_v[off + r, :]
        for d in range(D // 16):
            sl = pl.ds(d * 16, 16)
            rows[r, sl] = rows[r, sl] * pvec
        return carry

    lax.fori_loop(0, HALF, body, 0)


def _combine_body(ys_hbm, pos_hbm, psel_hbm, out_hbm, idx0, idx1, pv_v,
                  rows0, rows1, semA, semB):
    base = _sc_wid() * CHUNK
    pltpu.sync_copy(pos_hbm.at[pl.ds(base, HALF)], idx0)
    pltpu.sync_copy(pos_hbm.at[pl.ds(base + HALF, HALF)], idx1)
    g0 = pltpu.async_copy(ys_hbm.at[idx0], rows0, semA)
    g1 = pltpu.async_copy(ys_hbm.at[idx1], rows1, semB)
    pltpu.sync_copy(psel_hbm.at[pl.ds(base, CHUNK)], pv_v)
    g0.wait()
    _scale_rows(rows0, pv_v, 0)
    w0 = pltpu.async_copy(rows0, out_hbm.at[pl.ds(base, HALF)], semA)
    g1.wait()
    _scale_rows(rows1, pv_v, HALF)
    w1 = pltpu.async_copy(rows1, out_hbm.at[pl.ds(base + HALF, HALF)], semB)
    w0.wait()
    w1.wait()


@functools.cache
def _get_combine():
    return pl.kernel(
        _combine_body,
        mesh=plsc.VectorSubcoreMesh(core_axis_name="c", subcore_axis_name="s",
                                    num_cores=2, num_subcores=16),
        out_type=jax.ShapeDtypeStruct((N, D), jnp.float32),
        scratch_types=[
            pltpu.VMEM((HALF,), jnp.int32),
            pltpu.VMEM((HALF,), jnp.int32),
            pltpu.VMEM((CHUNK, 16), jnp.float32),
            pltpu.VMEM((HALF, D), jnp.float32),
            pltpu.VMEM((HALF, D), jnp.float32),
            pltpu.SemaphoreType.DMA,
            pltpu.SemaphoreType.DMA,
        ],
    )


@jax.jit
def kernel(x, input_ids, attention_mask, Wg, W1, b1, W2, b2):
    del input_ids, attention_mask
    bsz, seq_len, dim = x.shape
    x2d = x.reshape(N, D)

    pos2d, psel2d, counts, bm2d, be2d, sa2d, bal = _router(x2d, Wg)
    pos = pos2d.reshape(N)
    xs = _get_dispatch()(x2d, pos)
    ys = _ffn(bm2d.reshape(NB), be2d.reshape(NB), sa2d.reshape(NB), xs,
              W1, b1, W2, b2)
    out2d = _get_combine()(ys, pos, psel2d)

    out = out2d.reshape(bsz, seq_len, dim)
    balance_loss = bal.reshape(())
    gate_load = counts.reshape(E)
    return out, balance_loss, gate_load
